# baseline (speedup 1.0000x reference)
import functools

import jax
import jax.numpy as jnp
from jax import lax
from jax.experimental import pallas as pl
from jax.experimental.pallas import tpu as pltpu
from jax.experimental.pallas import tpu_sc as plsc

B = 16384
D = 64
NC = 2
NS = 16
NW = NC * NS
ROWS_PER_W = B // NW
CHUNK = 128
NCHUNK = ROWS_PER_W // CHUNK

_L = 16
_ND = D // _L


def _rsqrt_guard(s):
    i = lax.bitcast_convert_type(s, jnp.int32)
    i = jnp.int32(0x5F3759DF) - lax.shift_right_logical(i, 1)
    y = lax.bitcast_convert_type(i, jnp.float32)
    half = s * jnp.float32(0.5)
    for _ in range(3):
        y = y * (jnp.float32(1.5) - half * y * y)
    return jnp.minimum(y, jnp.float32(1e12))


_DNUMS = lax.GatherDimensionNumbers(
    offset_dims=(), collapsed_slice_dims=(0,), start_index_map=(0,))


def _shuffle(v, perm):
    return lax.gather(v, perm, _DNUMS, slice_sizes=(1,),
                      mode=lax.GatherScatterMode.PROMISE_IN_BOUNDS)


def _sum16(v):
    lanes = lax.iota(jnp.int32, _L)
    for k in (1, 2, 4, 8):
        perm = jnp.reshape(lanes ^ k, (_L, 1))
        v = v + _shuffle(v, perm)
    return v


def _body(h_idx_hbm, r_idx_hbm, t_idx_hbm, ent_hbm, rel_hbm, nrm_hbm,
          out_hbm, hidx_v, ridx_v, tidx_v, h_v, r_v, t_v, n_v, out_v, sem):
    wid = lax.axis_index("s") * NC + lax.axis_index("c")

    def chunk_body(c, carry):
        base = wid * ROWS_PER_W + c * CHUNK
        pltpu.sync_copy(h_idx_hbm.at[pl.ds(base, CHUNK)], hidx_v)
        pltpu.sync_copy(r_idx_hbm.at[pl.ds(base, CHUNK)], ridx_v)
        pltpu.sync_copy(t_idx_hbm.at[pl.ds(base, CHUNK)], tidx_v)
        d1 = pltpu.async_copy(ent_hbm.at[hidx_v], h_v, sem)
        d2 = pltpu.async_copy(rel_hbm.at[ridx_v], r_v, sem)
        d3 = pltpu.async_copy(ent_hbm.at[tidx_v], t_v, sem)
        d4 = pltpu.async_copy(nrm_hbm.at[ridx_v], n_v, sem)
        d1.wait()
        d2.wait()
        d3.wait()
        d4.wait()

        lanes = lax.iota(jnp.int32, _L)

        def row_body(i, svec):
            h = [h_v[i, pl.ds(k * _L, _L)] for k in range(_ND)]
            u = [n_v[i, pl.ds(k * _L, _L)] for k in range(_ND)]
            t = [t_v[i, pl.ds(k * _L, _L)] for k in range(_ND)]
            r = [r_v[i, pl.ds(k * _L, _L)] for k in range(_ND)]

            uu = _sum16(sum(u[k] * u[k] for k in range(_ND)))
            hu = _sum16(sum(h[k] * u[k] for k in range(_ND)))
            tu = _sum16(sum(t[k] * u[k] for k in range(_ND)))
            inv_den = jnp.float32(1.0) / jnp.maximum(uu, jnp.float32(1e-24))
            ah = hu * inv_den
            at = tu * inv_den
            hp = [h[k] - ah * u[k] for k in range(_ND)]
            tp = [t[k] - at * u[k] for k in range(_ND)]

            hh = _sum16(sum(hp[k] * hp[k] for k in range(_ND)))
            rr = _sum16(sum(r[k] * r[k] for k in range(_ND)))
            tt = _sum16(sum(tp[k] * tp[k] for k in range(_ND)))
            ih = _rsqrt_guard(hh)
            ir = _rsqrt_guard(rr)
            it = _rsqrt_guard(tt)

            sc = _sum16(sum(
                jnp.abs(hp[k] * ih + r[k] * ir - tp[k] * it)
                for k in range(_ND)))
            lane = lax.rem(i, _L)
            svec = jnp.where(lanes == lane, sc, svec)

            @pl.when(lane == _L - 1)
            def _():
                out_v[pl.ds(i - (_L - 1), _L)] = svec

            return svec

        lax.fori_loop(0, CHUNK, row_body, jnp.zeros((_L,), jnp.float32),
                      unroll=False)
        pltpu.sync_copy(out_v, out_hbm.at[pl.ds(base, CHUNK)])
        return carry

    lax.fori_loop(0, NCHUNK, chunk_body, 0, unroll=False)


@jax.jit
def _transh_sc(h_idx, r_idx, t_idx, entity_emb, relation_emb, norm_vec):
    mesh = plsc.VectorSubcoreMesh(core_axis_name="c", subcore_axis_name="s")
    return pl.kernel(
        _body,
        out_type=jax.ShapeDtypeStruct((B,), jnp.float32),
        mesh=mesh,
        scratch_types=[
            pltpu.VMEM((CHUNK,), jnp.int32),
            pltpu.VMEM((CHUNK,), jnp.int32),
            pltpu.VMEM((CHUNK,), jnp.int32),
            pltpu.VMEM((CHUNK, D), jnp.float32),
            pltpu.VMEM((CHUNK, D), jnp.float32),
            pltpu.VMEM((CHUNK, D), jnp.float32),
            pltpu.VMEM((CHUNK, D), jnp.float32),
            pltpu.VMEM((CHUNK,), jnp.float32),
            pltpu.SemaphoreType.DMA,
        ],
        compiler_params=pltpu.CompilerParams(use_tc_tiling_on_sc=False),
    )(h_idx, r_idx, t_idx, entity_emb, relation_emb, norm_vec)


def kernel(triplet_idx, entity_emb, relation_emb, norm_vec):
    cols = triplet_idx.T
    h_idx = cols[0]
    r_idx = cols[1]
    t_idx = cols[2]
    return _transh_sc(h_idx, r_idx, t_idx, entity_emb, relation_emb, norm_vec)
